# baseline (speedup 1.0000x reference)
import functools

import jax
import jax.numpy as jnp
from jax import lax
from jax.experimental import pallas as pl
from jax.experimental.pallas import tpu as pltpu
from jax.experimental.pallas import tpu_sc as plsc

_EPS = 1e-12

_NUM_CORES = 2
_NUM_SUBCORES = 16
_NUM_WORKERS = _NUM_CORES * _NUM_SUBCORES
_IDX_CHUNK = 128


def _sc_gather(table, ids):
    batch, seq = ids.shape
    n_idx = batch * seq
    depth = table.shape[1]
    per_w = n_idx // _NUM_WORKERS
    n_chunks = per_w // _IDX_CHUNK
    w_per_row = seq // per_w
    mesh = plsc.VectorSubcoreMesh(core_axis_name="c", subcore_axis_name="s")

    @functools.partial(
        pl.kernel,
        mesh=mesh,
        out_type=jax.ShapeDtypeStruct((n_idx, depth), jnp.float32),
        scratch_types=[
            pltpu.VMEM((per_w,), jnp.int32),
            pltpu.VMEM((per_w, depth), jnp.float32),
            pltpu.SemaphoreType.DMA,
        ],
    )
    def k(table_hbm, idx_hbm, out_hbm, idx_v, rows_v, sem):
        wid = lax.axis_index("s") * _NUM_CORES + lax.axis_index("c")
        row = wid // w_per_row
        col0 = (wid % w_per_row) * per_w
        pltpu.sync_copy(idx_hbm.at[row, pl.ds(col0, per_w)], idx_v)
        copies = [
            pltpu.async_copy(
                table_hbm.at[idx_v.at[pl.ds(j * _IDX_CHUNK, _IDX_CHUNK)]],
                rows_v.at[pl.ds(j * _IDX_CHUNK, _IDX_CHUNK)],
                sem,
            )
            for j in range(n_chunks)
        ]
        for c in copies:
            c.wait()
        pltpu.sync_copy(rows_v, out_hbm.at[pl.ds(wid * per_w, per_w)])

    return k(table, ids)


def _tc_add_ln(gathered3, pos_emb, token_type_ids, type_emb, gamma, beta, rows_per_step):
    batch, seq, hidden = gathered3.shape
    n_blk = batch // rows_per_step
    r = rows_per_step

    def body(g_ref, p_ref, tt_ref, te_ref, ga_ref, be_ref, o_ref):
        i = pl.program_id(0)
        x = g_ref[...] + p_ref[...]
        tt = jnp.stack([tt_ref[i * r + j, :] for j in range(r)])
        f = tt.astype(jnp.float32)[:, :, None]
        t0 = te_ref[0:1, :][None]
        t1 = te_ref[1:2, :][None]
        x = x + t0 + f * (t1 - t0)
        x2 = x.reshape(r * seq, hidden)
        w = jnp.full((hidden, hidden), 1.0 / hidden, dtype=jnp.bfloat16)
        xb = x2.astype(jnp.bfloat16)
        dn = (((1,), (0,)), ((), ()))
        mean = lax.dot_general(xb, w, dn, preferred_element_type=jnp.float32)
        exx = lax.dot_general(xb * xb, w, dn, preferred_element_type=jnp.float32)
        var = exx - mean * mean
        inv = lax.rsqrt(var + _EPS)
        o = (x2 - mean) * inv * ga_ref[...] + be_ref[...]
        o_ref[...] = o.reshape(r, seq, hidden)

    return pl.pallas_call(
        body,
        grid=(n_blk,),
        in_specs=[
            pl.BlockSpec((r, seq, hidden), lambda i: (i, 0, 0)),
            pl.BlockSpec((1, seq, hidden), lambda i: (0, 0, 0)),
            pl.BlockSpec((batch, seq), lambda i: (0, 0)),
            pl.BlockSpec((2, hidden), lambda i: (0, 0)),
            pl.BlockSpec((1, hidden), lambda i: (0, 0)),
            pl.BlockSpec((1, hidden), lambda i: (0, 0)),
        ],
        out_specs=pl.BlockSpec((r, seq, hidden), lambda i: (i, 0, 0)),
        out_shape=jax.ShapeDtypeStruct((batch, seq, hidden), jnp.float32),
    )(gathered3, pos_emb[None], token_type_ids, type_emb, gamma, beta)


def kernel(input_ids, token_type_ids, word_emb, pos_emb, type_emb, ln_gamma, ln_beta):
    batch, seq = input_ids.shape
    hidden = word_emb.shape[1]
    n_rows = batch * seq

    gathered = _sc_gather(word_emb, input_ids.astype(jnp.int32))

    return _tc_add_ln(
        gathered.reshape(batch, seq, hidden),
        pos_emb,
        token_type_ids.astype(jnp.int32),
        type_emb,
        ln_gamma.reshape(1, hidden),
        ln_beta.reshape(1, hidden),
        rows_per_step=2,
    )


# speedup vs baseline: 1.0124x; 1.0124x over previous
import functools

import jax
import jax.numpy as jnp
from jax import lax
from jax.experimental import pallas as pl
from jax.experimental.pallas import tpu as pltpu
from jax.experimental.pallas import tpu_sc as plsc

_EPS = 1e-12

_NUM_CORES = 2
_NUM_SUBCORES = 16
_NUM_WORKERS = _NUM_CORES * _NUM_SUBCORES
_IDX_CHUNK = 128


def _sc_gather(table, ids):
    batch, seq = ids.shape
    n_idx = batch * seq
    depth = table.shape[1]
    per_w = n_idx // _NUM_WORKERS
    n_chunks = per_w // _IDX_CHUNK
    w_per_row = seq // per_w
    mesh = plsc.VectorSubcoreMesh(core_axis_name="c", subcore_axis_name="s")

    @functools.partial(
        pl.kernel,
        mesh=mesh,
        out_type=jax.ShapeDtypeStruct((n_idx, depth), jnp.float32),
        scratch_types=[
            pltpu.VMEM((per_w,), jnp.int32),
            pltpu.VMEM((per_w, depth), jnp.float32),
            pltpu.SemaphoreType.DMA,
            pltpu.SemaphoreType.DMA,
        ],
    )
    def k(table_hbm, idx_hbm, out_hbm, idx_v, rows_v, gsem, wsem):
        wid = lax.axis_index("s") * _NUM_CORES + lax.axis_index("c")
        row = wid // w_per_row
        col0 = (wid % w_per_row) * per_w
        pltpu.sync_copy(idx_hbm.at[row, pl.ds(col0, per_w)], idx_v)
        gathers = [
            pltpu.async_copy(
                table_hbm.at[idx_v.at[pl.ds(j * _IDX_CHUNK, _IDX_CHUNK)]],
                rows_v.at[pl.ds(j * _IDX_CHUNK, _IDX_CHUNK)],
                gsem,
            )
            for j in range(n_chunks)
        ]
        writebacks = []
        for j in range(n_chunks):
            gathers[j].wait()
            writebacks.append(
                pltpu.async_copy(
                    rows_v.at[pl.ds(j * _IDX_CHUNK, _IDX_CHUNK)],
                    out_hbm.at[pl.ds(wid * per_w + j * _IDX_CHUNK, _IDX_CHUNK)],
                    wsem,
                )
            )
        for w in writebacks:
            w.wait()

    return k(table, ids)


def _tc_add_ln(gathered3, pos_emb, token_type_ids, type_emb, gamma, beta, rows_per_step):
    batch, seq, hidden = gathered3.shape
    n_blk = batch // rows_per_step
    r = rows_per_step

    def body(g_ref, p_ref, tt_ref, te_ref, ga_ref, be_ref, o_ref):
        i = pl.program_id(0)
        x = g_ref[...] + p_ref[...]
        tt = jnp.stack([tt_ref[i * r + j, :] for j in range(r)])
        f = tt.astype(jnp.float32)[:, :, None]
        t0 = te_ref[0:1, :][None]
        t1 = te_ref[1:2, :][None]
        x = x + t0 + f * (t1 - t0)
        x2 = x.reshape(r * seq, hidden)
        w = jnp.full((hidden, hidden), 1.0 / hidden, dtype=jnp.bfloat16)
        xb = x2.astype(jnp.bfloat16)
        dn = (((1,), (0,)), ((), ()))
        mean = lax.dot_general(xb, w, dn, preferred_element_type=jnp.float32)
        exx = lax.dot_general(xb * xb, w, dn, preferred_element_type=jnp.float32)
        var = exx - mean * mean
        inv = lax.rsqrt(var + _EPS)
        o = (x2 - mean) * inv * ga_ref[...] + be_ref[...]
        o_ref[...] = o.reshape(r, seq, hidden)

    return pl.pallas_call(
        body,
        grid=(n_blk,),
        in_specs=[
            pl.BlockSpec((r, seq, hidden), lambda i: (i, 0, 0)),
            pl.BlockSpec((1, seq, hidden), lambda i: (0, 0, 0)),
            pl.BlockSpec((batch, seq), lambda i: (0, 0)),
            pl.BlockSpec((2, hidden), lambda i: (0, 0)),
            pl.BlockSpec((1, hidden), lambda i: (0, 0)),
            pl.BlockSpec((1, hidden), lambda i: (0, 0)),
        ],
        out_specs=pl.BlockSpec((r, seq, hidden), lambda i: (i, 0, 0)),
        out_shape=jax.ShapeDtypeStruct((batch, seq, hidden), jnp.float32),
    )(gathered3, pos_emb[None], token_type_ids, type_emb, gamma, beta)


def kernel(input_ids, token_type_ids, word_emb, pos_emb, type_emb, ln_gamma, ln_beta):
    batch, seq = input_ids.shape
    hidden = word_emb.shape[1]
    n_rows = batch * seq

    gathered = _sc_gather(word_emb, input_ids.astype(jnp.int32))

    return _tc_add_ln(
        gathered.reshape(batch, seq, hidden),
        pos_emb,
        token_type_ids.astype(jnp.int32),
        type_emb,
        ln_gamma.reshape(1, hidden),
        ln_beta.reshape(1, hidden),
        rows_per_step=2,
    )
